# baseline (speedup 1.0000x reference)
import functools

import jax
import jax.numpy as jnp
from jax import lax
from jax.experimental import pallas as pl
from jax.experimental.pallas import tpu as pltpu
from jax.experimental.pallas import tpu_sc as plsc

_LANES = 16
_CHUNK = 128


@functools.lru_cache(maxsize=None)
def _build(n_flat, seq, vocab, dim, pos_rows, n_workers):
    per_w = n_flat // n_workers
    n_chunks = per_w // _CHUNK
    mesh = plsc.VectorSubcoreMesh(core_axis_name="c", subcore_axis_name="s")
    info = plsc.get_sparse_core_info()
    num_cores = info.num_cores

    @functools.partial(
        pl.kernel,
        mesh=mesh,
        compiler_params=pltpu.CompilerParams(use_tc_tiling_on_sc=False),
        out_type=jax.ShapeDtypeStruct((n_flat, dim), jnp.float32),
        scratch_types=[
            pltpu.VMEM((_CHUNK,), jnp.int32),
            pltpu.VMEM((_CHUNK, dim), jnp.float32),
            pltpu.VMEM((pos_rows, dim), jnp.float32),
            pltpu.SemaphoreType.DMA,
        ],
    )
    def emb(table_hbm, idx_hbm, pos_hbm, out_hbm, idx_v, rows_v, pos_v, sem):
        wid = lax.axis_index("s") * num_cores + lax.axis_index("c")
        base = wid * per_w
        pltpu.sync_copy(pos_hbm, pos_v)

        def chunk_body(g, carry):
            off = base + g * _CHUNK
            pltpu.sync_copy(idx_hbm.at[pl.ds(off, _CHUNK)], idx_v)
            pltpu.async_copy(table_hbm.at[idx_v], rows_v, sem).wait()
            pbase = lax.rem(g * _CHUNK, seq)

            def row_body(r, c2):
                pr = pbase + r
                for cc in range(dim // _LANES):
                    sl = pl.ds(cc * _LANES, _LANES)
                    rows_v[r, sl] = rows_v[r, sl] + pos_v[pr, sl]
                return c2

            lax.fori_loop(0, _CHUNK, row_body, 0)
            pltpu.sync_copy(rows_v, out_hbm.at[pl.ds(off, _CHUNK)])
            return carry

        lax.fori_loop(0, n_chunks, chunk_body, 0)

    return emb


def kernel(input_ids, table, pos_encoding):
    b, s = input_ids.shape
    v, d = table.shape
    ids_flat = input_ids.reshape(-1).astype(jnp.int32)
    n_flat = b * s
    pos_seq = pos_encoding[:s]
    pos_ext = jnp.concatenate([pos_seq, pos_seq[:_CHUNK]], axis=0)
    emb = _build(n_flat, s, v, d, pos_ext.shape[0], 32)
    out = emb(table, ids_flat, pos_ext)
    return out.reshape(b, s, d)


# speedup vs baseline: 1.4688x; 1.4688x over previous
import functools

import jax
import jax.numpy as jnp
from jax import lax
from jax.experimental import pallas as pl
from jax.experimental.pallas import tpu as pltpu
from jax.experimental.pallas import tpu_sc as plsc

_IDX = 128
_SUB = 4
_CHUNK = _IDX * _SUB


@functools.lru_cache(maxsize=None)
def _build(n_flat, seq, vocab, dim, pos_rows, n_workers):
    per_w = n_flat // n_workers
    n_chunks = per_w // _CHUNK
    rows_per_w = per_w // _IDX
    mesh = plsc.VectorSubcoreMesh(core_axis_name="c", subcore_axis_name="s")
    info = plsc.get_sparse_core_info()
    num_cores = info.num_cores

    @functools.partial(
        pl.kernel,
        mesh=mesh,
        compiler_params=pltpu.CompilerParams(use_tc_tiling_on_sc=False),
        out_type=jax.ShapeDtypeStruct((n_flat, dim), jnp.float32),
        scratch_types=[
            pltpu.VMEM((_SUB, _IDX), jnp.int32),
            pltpu.VMEM((_CHUNK, dim), jnp.float32),
            pltpu.VMEM((pos_rows, dim), jnp.float32),
            pltpu.SemaphoreType.DMA,
        ],
    )
    def emb(table_hbm, idx_hbm, pos_hbm, out_hbm, idx_v, rows_v, pos_v, sem):
        wid = lax.axis_index("s") * num_cores + lax.axis_index("c")
        base = wid * per_w
        idx_row_base = wid * rows_per_w
        pltpu.sync_copy(pos_hbm, pos_v)

        def chunk_body(g, carry):
            off = base + g * _CHUNK
            pltpu.sync_copy(idx_hbm.at[pl.ds(idx_row_base + g * _SUB, _SUB)],
                            idx_v)
            pbase = lax.rem(g * _CHUNK, seq)

            @plsc.parallel_loop(0, _CHUNK, 1, unroll=8)
            def _prefill(r):
                pr = pbase + r
                for cc in range(dim // 16):
                    sl = pl.ds(cc * 16, 16)
                    rows_v[r, sl] = pos_v[pr, sl]
            descs = [
                pltpu.async_copy(
                    table_hbm.at[idx_v.at[j]],
                    rows_v.at[pl.ds(j * _IDX, _IDX)],
                    sem,
                    add=True,
                )
                for j in range(_SUB)
            ]
            for d in descs:
                d.wait()
            pltpu.sync_copy(rows_v, out_hbm.at[pl.ds(off, _CHUNK)])
            return carry

        lax.fori_loop(0, n_chunks, chunk_body, 0)

    return emb


def kernel(input_ids, table, pos_encoding):
    b, s = input_ids.shape
    v, d = table.shape
    n_flat = b * s
    ids_2d = input_ids.reshape(n_flat // _IDX, _IDX).astype(jnp.int32)
    pos_seq = pos_encoding[:s]
    reps = -(-(_CHUNK) // s) + 1
    pos_ext = jnp.concatenate([pos_seq] * reps, axis=0)[: s + _CHUNK]
    emb = _build(n_flat, s, v, d, pos_ext.shape[0], 32)
    out = emb(table, ids_2d, pos_ext)
    return out.reshape(b, s, d)


# speedup vs baseline: 1.5261x; 1.0390x over previous
import functools

import jax
import jax.numpy as jnp
from jax import lax
from jax.experimental import pallas as pl
from jax.experimental.pallas import tpu as pltpu
from jax.experimental.pallas import tpu_sc as plsc

_IDX = 128
_SUB = 4
_CHUNK = _IDX * _SUB
_NBUF = 2


@functools.lru_cache(maxsize=None)
def _build(n_flat, seq, vocab, dim, pos_rows, n_workers):
    per_w = n_flat // n_workers
    n_chunks = per_w // _CHUNK
    n_outer = n_chunks // _NBUF
    rows_per_w = per_w // _IDX
    mesh = plsc.VectorSubcoreMesh(core_axis_name="c", subcore_axis_name="s")
    info = plsc.get_sparse_core_info()
    num_cores = info.num_cores

    @functools.partial(
        pl.kernel,
        mesh=mesh,
        compiler_params=pltpu.CompilerParams(use_tc_tiling_on_sc=False),
        out_type=jax.ShapeDtypeStruct((n_flat, dim), jnp.float32),
        scratch_types=[
            pltpu.VMEM((_NBUF, _SUB, _IDX), jnp.int32),
            pltpu.VMEM((_NBUF, _CHUNK, dim), jnp.float32),
            pltpu.VMEM((pos_rows, dim), jnp.float32),
            pltpu.SemaphoreType.DMA,
            pltpu.SemaphoreType.DMA,
            pltpu.SemaphoreType.DMA,
            pltpu.SemaphoreType.DMA,
        ],
    )
    def emb(table_hbm, idx_hbm, pos_hbm, out_hbm, idx_v, rows_v, pos_v,
            gsem0, gsem1, ssem0, ssem1):
        gsem = (gsem0, gsem1)
        ssem = (ssem0, ssem1)
        wid = lax.axis_index("s") * num_cores + lax.axis_index("c")
        base = wid * per_w
        idx_row_base = wid * rows_per_w
        pltpu.sync_copy(pos_hbm, pos_v)

        def out_slice(chunk_id):
            return out_hbm.at[pl.ds(base + chunk_id * _CHUNK, _CHUNK)]

        def stage(chunk_id, b):
            pltpu.sync_copy(
                idx_hbm.at[pl.ds(idx_row_base + chunk_id * _SUB, _SUB)],
                idx_v.at[b])
            pbase = lax.rem(chunk_id * _CHUNK, seq)

            @plsc.parallel_loop(0, _CHUNK, 1, unroll=8)
            def _prefill(r):
                pr = pbase + r
                for cc in range(dim // 16):
                    sl = pl.ds(cc * 16, 16)
                    rows_v[b, r, sl] = pos_v[pr, sl]

            for j in range(_SUB):
                pltpu.async_copy(
                    table_hbm.at[idx_v.at[b, j]],
                    rows_v.at[b, pl.ds(j * _IDX, _IDX)],
                    gsem[b],
                    add=True,
                )

        def wait_gathers(chunk_id, b):
            pltpu.make_async_copy(out_slice(chunk_id), rows_v.at[b],
                                  gsem[b]).wait()

        def fire_scatter(chunk_id, b):
            pltpu.async_copy(rows_v.at[b], out_slice(chunk_id), ssem[b])

        def wait_scatter(chunk_id, b):
            pltpu.make_async_copy(rows_v.at[b], out_slice(chunk_id),
                                  ssem[b]).wait()

        def outer(g, carry):
            for b in range(_NBUF):
                cid = g * _NBUF + b
                @pl.when(g >= 1)
                def _():
                    wait_scatter(cid - _NBUF, b)

                stage(cid, b)

                o = 1 - b
                pcid = cid - 1

                @pl.when(cid >= 1)
                def _():
                    wait_gathers(pcid, o)
                    fire_scatter(pcid, o)

            return carry

        lax.fori_loop(0, n_outer, outer, 0)

        last = n_chunks - 1
        bl = last % _NBUF
        wait_gathers(last, bl)
        fire_scatter(last, bl)
        wait_scatter(last - 1, 1 - bl)
        wait_scatter(last, bl)

    return emb


def kernel(input_ids, table, pos_encoding):
    b, s = input_ids.shape
    v, d = table.shape
    n_flat = b * s
    ids_2d = input_ids.reshape(n_flat // _IDX, _IDX).astype(jnp.int32)
    pos_seq = pos_encoding[:s]
    reps = -(-(_CHUNK) // s) + 1
    pos_ext = jnp.concatenate([pos_seq] * reps, axis=0)[: s + _CHUNK]
    emb = _build(n_flat, s, v, d, pos_ext.shape[0], 32)
    out = emb(table, ids_2d, pos_ext)
    return out.reshape(b, s, d)
